# baseline (speedup 1.0000x reference)
import functools

import jax
import jax.numpy as jnp
from jax import lax
from jax.experimental import pallas as pl
from jax.experimental.pallas import tpu as pltpu
from jax.experimental.pallas import tpu_sc as plsc

_DIM = 64
_NEMB = 1024
_ROWS = 9216
_BLK = 512
_NSTEP = _ROWS // _BLK

_NW = 32
_BPW = _ROWS // _NW
_CH = 96


def _tc_body(x_ref, e_ref, en_ref, ind_ref, loss_ref):
    i = pl.program_id(0)
    e = e_ref[...]
    norm = jnp.sqrt(jnp.sum(e * e, axis=1, keepdims=True))
    en = e / norm
    colsq = jnp.sum(en * en, axis=1, keepdims=True)

    x = x_ref[...]
    xsq = jnp.sum(x * x, axis=1, keepdims=True)
    dot = lax.dot_general(x, en, (((1,), (1,)), ((), ())),
                          preferred_element_type=jnp.float32)
    nd = (2.0 * dot - xsq) - colsq.T
    m = jnp.max(nd, axis=1, keepdims=True)
    iota = lax.broadcasted_iota(jnp.int32, nd.shape, 1)
    ind = jnp.min(jnp.where(nd == m, iota, jnp.int32(2 ** 30)),
                  axis=1, keepdims=True)
    ind_ref[0] = ind
    sdist = -jnp.sum(m)

    @pl.when(i == 0)
    def _():
        en_ref[...] = en
        ssum = jnp.sum(en, axis=0, keepdims=True)
        entropy = (2.0 * _NEMB * jnp.sum(colsq)
                   - 2.0 * jnp.sum(ssum * ssum))
        loss_ref[0, 0] = (sdist * (1.0 / (_ROWS * _DIM))
                          - entropy * (1.0 / (_NEMB * _NEMB)))

    @pl.when(i != 0)
    def _():
        loss_ref[0, 0] += sdist * (1.0 / (_ROWS * _DIM))


_tc_call = pl.pallas_call(
    _tc_body,
    grid=(_NSTEP,),
    in_specs=[
        pl.BlockSpec((_BLK, _DIM), lambda i: (i, 0)),
        pl.BlockSpec((_NEMB, _DIM), lambda i: (0, 0)),
    ],
    out_specs=[
        pl.BlockSpec((_NEMB, _DIM), lambda i: (0, 0)),
        pl.BlockSpec((1, _BLK, 1), lambda i: (i, 0, 0)),
        pl.BlockSpec((1, 1), lambda i: (0, 0),
                     memory_space=pltpu.SMEM),
    ],
    out_shape=[
        jax.ShapeDtypeStruct((_NEMB, _DIM), jnp.float32),
        jax.ShapeDtypeStruct((_NSTEP, _BLK, 1), jnp.int32),
        jax.ShapeDtypeStruct((1, 1), jnp.float32),
    ],
    compiler_params=pltpu.CompilerParams(
        dimension_semantics=("arbitrary",)),
)


def _sc_gather(table, idx):
    mesh = plsc.VectorSubcoreMesh(core_axis_name="c", subcore_axis_name="s")

    @functools.partial(
        pl.kernel, mesh=mesh,
        out_type=jax.ShapeDtypeStruct((_ROWS, _DIM), jnp.float32),
        scratch_types=[
            pltpu.VMEM((_BPW // _CH, _CH), jnp.int32),
            pltpu.VMEM((_BPW, _DIM), jnp.float32),
            pltpu.SemaphoreType.DMA,
        ],
        compiler_params=pltpu.CompilerParams(use_tc_tiling_on_sc=False),
    )
    def k(table_hbm, idx_hbm, out_hbm, idx_v, rows_v, sem):
        wid = lax.axis_index("s") * 2 + lax.axis_index("c")
        base = wid * _BPW
        copies = []
        for c in range(_BPW // _CH):
            pltpu.sync_copy(idx_hbm.at[pl.ds(base + c * _CH, _CH)],
                            idx_v.at[c])
            copies.append(
                pltpu.async_copy(table_hbm.at[idx_v.at[c]],
                                 rows_v.at[pl.ds(c * _CH, _CH)], sem))
        for cp in copies:
            cp.wait()
        pltpu.sync_copy(rows_v, out_hbm.at[pl.ds(base, _BPW)])

    return k(table, idx)


def kernel(input, embedding):
    x = input.reshape(_ROWS, _DIM)
    en, ind3, loss = _tc_call(x, embedding)
    ind = ind3.reshape(_ROWS)
    q = _sc_gather(en, ind)
    return (q.reshape(input.shape), loss[0, 0], ind)


# speedup vs baseline: 1.2106x; 1.2106x over previous
import functools

import jax
import jax.numpy as jnp
from jax import lax
from jax.experimental import pallas as pl
from jax.experimental.pallas import tpu as pltpu
from jax.experimental.pallas import tpu_sc as plsc

_DIM = 64
_NEMB = 1024
_ROWS = 9216
_BLK = 1152
_NSTEP = _ROWS // _BLK

_NW = 32
_BPW = _ROWS // _NW
_CH = 96


def _tc_body(x_ref, e_ref, en_ref, ind_ref, loss_ref):
    i = pl.program_id(0)
    e = e_ref[...]
    norm = jnp.sqrt(jnp.sum(e * e, axis=1, keepdims=True))
    en = e / norm
    colsq = jnp.sum(en * en, axis=1, keepdims=True)
    en2 = en + en

    x = x_ref[...]
    xsq = jnp.sum(x * x, axis=1, keepdims=True)
    dot2 = lax.dot_general(x, en2, (((1,), (1,)), ((), ())),
                           preferred_element_type=jnp.float32)
    nd = (dot2 - xsq) - colsq.T
    m = jnp.max(nd, axis=1, keepdims=True)
    iota = lax.broadcasted_iota(jnp.int32, nd.shape, 1)
    ind = jnp.min(jnp.where(nd == m, iota, jnp.int32(2 ** 30)),
                  axis=1, keepdims=True)
    ind_ref[0] = ind.T
    sdist = -jnp.sum(m)

    @pl.when(i == 0)
    def _():
        en_ref[...] = en
        ssum = jnp.sum(en, axis=0, keepdims=True)
        entropy = (2.0 * _NEMB * jnp.sum(colsq)
                   - 2.0 * jnp.sum(ssum * ssum))
        loss_ref[0, 0] = (sdist * (1.0 / (_ROWS * _DIM))
                          - entropy * (1.0 / (_NEMB * _NEMB)))

    @pl.when(i != 0)
    def _():
        loss_ref[0, 0] += sdist * (1.0 / (_ROWS * _DIM))


_tc_call = pl.pallas_call(
    _tc_body,
    grid=(_NSTEP,),
    in_specs=[
        pl.BlockSpec((_BLK, _DIM), lambda i: (i, 0)),
        pl.BlockSpec((_NEMB, _DIM), lambda i: (0, 0)),
    ],
    out_specs=[
        pl.BlockSpec((_NEMB, _DIM), lambda i: (0, 0)),
        pl.BlockSpec((1, 1, _BLK), lambda i: (i, 0, 0)),
        pl.BlockSpec((1, 1), lambda i: (0, 0),
                     memory_space=pltpu.SMEM),
    ],
    out_shape=[
        jax.ShapeDtypeStruct((_NEMB, _DIM), jnp.float32),
        jax.ShapeDtypeStruct((_NSTEP, 1, _BLK), jnp.int32),
        jax.ShapeDtypeStruct((1, 1), jnp.float32),
    ],
    compiler_params=pltpu.CompilerParams(
        dimension_semantics=("arbitrary",)),
)


def _sc_gather(table, idx):
    mesh = plsc.VectorSubcoreMesh(core_axis_name="c", subcore_axis_name="s")

    @functools.partial(
        pl.kernel, mesh=mesh,
        out_type=jax.ShapeDtypeStruct((_ROWS, _DIM), jnp.float32),
        scratch_types=[
            pltpu.VMEM((_BPW // _CH, _CH), jnp.int32),
            pltpu.VMEM((_BPW, _DIM), jnp.float32),
            pltpu.SemaphoreType.DMA,
        ],
        compiler_params=pltpu.CompilerParams(use_tc_tiling_on_sc=False),
    )
    def k(table_hbm, idx_hbm, out_hbm, idx_v, rows_v, sem):
        wid = lax.axis_index("s") * 2 + lax.axis_index("c")
        base = wid * _BPW
        pltpu.sync_copy(idx_hbm.at[wid], idx_v)
        copies = []
        for c in range(_BPW // _CH):
            copies.append(
                pltpu.async_copy(table_hbm.at[idx_v.at[c]],
                                 rows_v.at[pl.ds(c * _CH, _CH)], sem))
        for cp in copies:
            cp.wait()
        pltpu.sync_copy(rows_v, out_hbm.at[pl.ds(base, _BPW)])

    return k(table, idx)


def kernel(input, embedding):
    x = input.reshape(_ROWS, _DIM)
    en, ind3, loss = _tc_call(x, embedding)
    ind = ind3.reshape(_ROWS)
    q = _sc_gather(en, ind.reshape(_NW, _BPW // _CH, _CH))
    return (q.reshape(input.shape), loss[0, 0], ind)


# speedup vs baseline: 1.2747x; 1.0530x over previous
import functools

import jax
import jax.numpy as jnp
from jax import lax
from jax.experimental import pallas as pl
from jax.experimental.pallas import tpu as pltpu
from jax.experimental.pallas import tpu_sc as plsc

_DIM = 64
_NEMB = 1024
_B0 = 16
_S0 = 576
_ROWS = _B0 * _S0
_BLK = 1152
_BB = _BLK // _S0
_NSTEP = _ROWS // _BLK

_NW = 32
_BPW = _ROWS // _NW
_CH = 96


def _tc_body(x_ref, e_ref, en_ref, ind_ref, loss_ref, en2_s, colsq_s):
    i = pl.program_id(0)

    @pl.when(i == 0)
    def _():
        e = e_ref[...]
        norm = jnp.sqrt(jnp.sum(e * e, axis=1, keepdims=True))
        en = e / norm
        colsq = jnp.sum(en * en, axis=1, keepdims=True)
        en_ref[...] = en
        en2_s[...] = en + en
        colsq_s[...] = colsq.T
        ssum = jnp.sum(en, axis=0, keepdims=True)
        entropy = (2.0 * _NEMB * jnp.sum(colsq)
                   - 2.0 * jnp.sum(ssum * ssum))
        loss_ref[0, 0] = -entropy * (1.0 / (_NEMB * _NEMB))

    x = x_ref[...].reshape(_BLK, _DIM)
    xsq = jnp.sum(x * x, axis=1, keepdims=True)
    dot2 = lax.dot_general(x, en2_s[...], (((1,), (1,)), ((), ())),
                           preferred_element_type=jnp.float32)
    nd = (dot2 - xsq) - colsq_s[...]
    m = jnp.max(nd, axis=1, keepdims=True)
    iota = lax.broadcasted_iota(jnp.int32, nd.shape, 1)
    ind = jnp.min(jnp.where(nd == m, iota, jnp.int32(2 ** 30)),
                  axis=1, keepdims=True)
    ind_ref[pl.ds(i * _BLK, _BLK)] = ind.T.reshape(_BLK)
    loss_ref[0, 0] += -jnp.sum(m) * (1.0 / (_ROWS * _DIM))


_tc_call = pl.pallas_call(
    _tc_body,
    grid=(_NSTEP,),
    in_specs=[
        pl.BlockSpec((_BB, _S0, _DIM), lambda i: (i, 0, 0)),
        pl.BlockSpec((_NEMB, _DIM), lambda i: (0, 0)),
    ],
    out_specs=[
        pl.BlockSpec((_NEMB, _DIM), lambda i: (0, 0)),
        pl.BlockSpec((_ROWS,), lambda i: (0,)),
        pl.BlockSpec((1, 1), lambda i: (0, 0),
                     memory_space=pltpu.SMEM),
    ],
    out_shape=[
        jax.ShapeDtypeStruct((_NEMB, _DIM), jnp.float32),
        jax.ShapeDtypeStruct((_ROWS,), jnp.int32),
        jax.ShapeDtypeStruct((1, 1), jnp.float32),
    ],
    scratch_shapes=[
        pltpu.VMEM((_NEMB, _DIM), jnp.float32),
        pltpu.VMEM((1, _NEMB), jnp.float32),
    ],
    compiler_params=pltpu.CompilerParams(
        dimension_semantics=("arbitrary",)),
)


def _sc_gather(table, idx):
    mesh = plsc.VectorSubcoreMesh(core_axis_name="c", subcore_axis_name="s")

    @functools.partial(
        pl.kernel, mesh=mesh,
        out_type=jax.ShapeDtypeStruct((_ROWS, _DIM), jnp.float32),
        scratch_types=[
            pltpu.VMEM((_BPW,), jnp.int32),
            pltpu.VMEM((_BPW, _DIM), jnp.float32),
            pltpu.SemaphoreType.DMA,
        ],
        compiler_params=pltpu.CompilerParams(use_tc_tiling_on_sc=False),
    )
    def k(table_hbm, idx_hbm, out_hbm, idx_v, rows_v, sem):
        wid = lax.axis_index("s") * 2 + lax.axis_index("c")
        base = wid * _BPW
        pltpu.sync_copy(idx_hbm.at[pl.ds(base, _BPW)], idx_v)
        copies = []
        for c in range(_BPW // _CH):
            copies.append(
                pltpu.async_copy(table_hbm.at[idx_v.at[pl.ds(c * _CH, _CH)]],
                                 rows_v.at[pl.ds(c * _CH, _CH)], sem))
        for cp in copies:
            cp.wait()
        pltpu.sync_copy(rows_v, out_hbm.at[pl.ds(base, _BPW)])

    return k(table, idx)


def kernel(input, embedding):
    en, ind, loss = _tc_call(input, embedding)
    q = _sc_gather(en, ind)
    return (q.reshape(input.shape), loss[0, 0], ind)


# speedup vs baseline: 1.4884x; 1.1677x over previous
import functools

import jax
import jax.numpy as jnp
from jax import lax
from jax.experimental import pallas as pl
from jax.experimental.pallas import tpu as pltpu
from jax.experimental.pallas import tpu_sc as plsc

_DIM = 64
_NEMB = 1024
_B0 = 16
_S0 = 576
_ROWS = _B0 * _S0
_BB = 2
_NSTEP = _B0 // _BB

_NW = 32
_BPW = _ROWS // _NW
_CH = 96


def _tc_body(x_ref, e_ref, enp_ref, ind_ref, loss_ref, en2_s, colsq_s):
    i = pl.program_id(0)

    @pl.when(i == 0)
    def _():
        e = e_ref[...]
        norm = jnp.sqrt(jnp.sum(e * e, axis=0, keepdims=True))
        en_t = e / norm
        en2_s[...] = en_t + en_t
        colsq_t = jnp.sum(en_t * en_t, axis=0, keepdims=True)
        colsq_s[...] = colsq_t.T
        enp_ref[...] = en_t.T
        ssum = jnp.sum(en_t, axis=1, keepdims=True)
        entropy = (2.0 * _NEMB * jnp.sum(colsq_t)
                   - 2.0 * jnp.sum(ssum * ssum))
        loss_ref[0, 0] = -entropy * (1.0 / (_NEMB * _NEMB))

    inds = []
    sdist = jnp.float32(0.0)
    for b in range(_BB):
        x = x_ref[b]
        xsq = jnp.sum(x * x, axis=0, keepdims=True)
        dot2 = lax.dot_general(en2_s[...], x, (((0,), (0,)), ((), ())),
                               preferred_element_type=jnp.float32)
        nd = (dot2 - xsq) - colsq_s[...]
        m = jnp.max(nd, axis=0, keepdims=True)
        iota = lax.broadcasted_iota(jnp.int32, nd.shape, 0)
        inds.append(jnp.min(jnp.where(nd == m, iota, jnp.int32(2 ** 30)),
                            axis=0, keepdims=True))
        sdist = sdist - jnp.sum(m)
    ind_blk = jnp.concatenate(inds, axis=1).reshape(_BB * _S0)
    ind_ref[pl.ds(i * (_BB * _S0), _BB * _S0)] = ind_blk
    loss_ref[0, 0] += sdist * (1.0 / (_ROWS * _DIM))


_tc_call = pl.pallas_call(
    _tc_body,
    grid=(_NSTEP,),
    in_specs=[
        pl.BlockSpec((_BB, _DIM, _S0), lambda i: (i, 0, 0)),
        pl.BlockSpec((_DIM, _NEMB), lambda i: (0, 0)),
    ],
    out_specs=[
        pl.BlockSpec((_NEMB, _DIM), lambda i: (0, 0)),
        pl.BlockSpec((_ROWS,), lambda i: (0,)),
        pl.BlockSpec((1, 1), lambda i: (0, 0),
                     memory_space=pltpu.SMEM),
    ],
    out_shape=[
        jax.ShapeDtypeStruct((_NEMB, _DIM), jnp.float32),
        jax.ShapeDtypeStruct((_ROWS,), jnp.int32),
        jax.ShapeDtypeStruct((1, 1), jnp.float32),
    ],
    scratch_shapes=[
        pltpu.VMEM((_DIM, _NEMB), jnp.float32),
        pltpu.VMEM((_NEMB, 1), jnp.float32),
    ],
    compiler_params=pltpu.CompilerParams(
        dimension_semantics=("arbitrary",)),
)


def _sc_gather(table, idx):
    mesh = plsc.VectorSubcoreMesh(core_axis_name="c", subcore_axis_name="s")

    @functools.partial(
        pl.kernel, mesh=mesh,
        out_type=jax.ShapeDtypeStruct((_ROWS, _DIM), jnp.float32),
        scratch_types=[
            pltpu.VMEM((_BPW,), jnp.int32),
            pltpu.VMEM((_BPW, _DIM), jnp.float32),
            pltpu.SemaphoreType.DMA,
        ],
        compiler_params=pltpu.CompilerParams(use_tc_tiling_on_sc=False),
    )
    def k(table_hbm, idx_hbm, out_hbm, idx_v, rows_v, sem):
        wid = lax.axis_index("s") * 2 + lax.axis_index("c")
        base = wid * _BPW
        pltpu.sync_copy(idx_hbm.at[pl.ds(base, _BPW)], idx_v)
        copies = []
        for c in range(_BPW // _CH):
            copies.append(
                pltpu.async_copy(table_hbm.at[idx_v.at[pl.ds(c * _CH, _CH)]],
                                 rows_v.at[pl.ds(c * _CH, _CH)], sem))
        for cp in copies:
            cp.wait()
        pltpu.sync_copy(rows_v, out_hbm.at[pl.ds(base, _BPW)])

    return k(table, idx)


def kernel(input, embedding):
    x_t = jnp.swapaxes(input, 1, 2)
    e_t = embedding.T
    table, ind, loss = _tc_call(x_t, e_t)
    q = _sc_gather(table, ind)
    return (q.reshape(_B0, _S0, _DIM), loss[0, 0], ind)


# speedup vs baseline: 1.5451x; 1.0381x over previous
import functools

import jax
import jax.numpy as jnp
from jax import lax
from jax.experimental import pallas as pl
from jax.experimental.pallas import tpu as pltpu
from jax.experimental.pallas import tpu_sc as plsc

_DIM = 64
_NEMB = 1024
_B0 = 16
_S0 = 576
_ROWS = _B0 * _S0
_BB = 8
_NSTEP = _B0 // _BB

_NW = 32
_BPW = _ROWS // _NW
_CH = 96


def _tc_body(x_ref, e_ref, enp_ref, ind_ref, loss_ref, en2_s, colsq_s):
    i = pl.program_id(0)

    @pl.when(i == 0)
    def _():
        e = e_ref[...]
        norm = jnp.sqrt(jnp.sum(e * e, axis=0, keepdims=True))
        en_t = e / norm
        en2_s[...] = en_t + en_t
        colsq_t = jnp.sum(en_t * en_t, axis=0, keepdims=True)
        colsq_s[...] = colsq_t.T
        enp_ref[...] = en_t.T
        ssum = jnp.sum(en_t, axis=1, keepdims=True)
        entropy = (2.0 * _NEMB * jnp.sum(colsq_t)
                   - 2.0 * jnp.sum(ssum * ssum))
        loss_ref[0, 0] = -entropy * (1.0 / (_NEMB * _NEMB))

    inds = []
    sdist = jnp.float32(0.0)
    for b in range(_BB):
        x = x_ref[b]
        xsq = jnp.sum(x * x, axis=0, keepdims=True)
        dot2 = lax.dot_general(en2_s[...], x, (((0,), (0,)), ((), ())),
                               preferred_element_type=jnp.float32)
        nd = (dot2 - xsq) - colsq_s[...]
        m = jnp.max(nd, axis=0, keepdims=True)
        iota = lax.broadcasted_iota(jnp.int32, nd.shape, 0)
        inds.append(jnp.min(jnp.where(nd == m, iota, jnp.int32(2 ** 30)),
                            axis=0, keepdims=True))
        sdist = sdist - jnp.sum(m)
    ind_blk = jnp.concatenate(inds, axis=1).reshape(_BB * _S0)
    ind_ref[pl.ds(i * (_BB * _S0), _BB * _S0)] = ind_blk
    loss_ref[0, 0] += sdist * (1.0 / (_ROWS * _DIM))


_tc_call = pl.pallas_call(
    _tc_body,
    grid=(_NSTEP,),
    in_specs=[
        pl.BlockSpec((_BB, _DIM, _S0), lambda i: (i, 0, 0)),
        pl.BlockSpec((_DIM, _NEMB), lambda i: (0, 0)),
    ],
    out_specs=[
        pl.BlockSpec((_NEMB, _DIM), lambda i: (0, 0)),
        pl.BlockSpec((_ROWS,), lambda i: (0,)),
        pl.BlockSpec((1, 1), lambda i: (0, 0),
                     memory_space=pltpu.SMEM),
    ],
    out_shape=[
        jax.ShapeDtypeStruct((_NEMB, _DIM), jnp.float32),
        jax.ShapeDtypeStruct((_ROWS,), jnp.int32),
        jax.ShapeDtypeStruct((1, 1), jnp.float32),
    ],
    scratch_shapes=[
        pltpu.VMEM((_DIM, _NEMB), jnp.float32),
        pltpu.VMEM((_NEMB, 1), jnp.float32),
    ],
    compiler_params=pltpu.CompilerParams(
        dimension_semantics=("arbitrary",)),
)


def _sc_gather(table, idx):
    mesh = plsc.VectorSubcoreMesh(core_axis_name="c", subcore_axis_name="s")

    @functools.partial(
        pl.kernel, mesh=mesh,
        out_type=jax.ShapeDtypeStruct((_ROWS, _DIM), jnp.float32),
        scratch_types=[
            pltpu.VMEM((_BPW,), jnp.int32),
            pltpu.VMEM((_BPW, _DIM), jnp.float32),
            pltpu.SemaphoreType.DMA,
        ],
        compiler_params=pltpu.CompilerParams(use_tc_tiling_on_sc=False),
    )
    def k(table_hbm, idx_hbm, out_hbm, idx_v, rows_v, sem):
        wid = lax.axis_index("s") * 2 + lax.axis_index("c")
        base = wid * _BPW
        pltpu.sync_copy(idx_hbm.at[pl.ds(base, _BPW)], idx_v)
        copies = []
        for c in range(_BPW // _CH):
            copies.append(
                pltpu.async_copy(table_hbm.at[idx_v.at[pl.ds(c * _CH, _CH)]],
                                 rows_v.at[pl.ds(c * _CH, _CH)], sem))
        for cp in copies:
            cp.wait()
        pltpu.sync_copy(rows_v, out_hbm.at[pl.ds(base, _BPW)])

    return k(table, idx)


def kernel(input, embedding):
    x_t = jnp.swapaxes(input, 1, 2)
    e_t = embedding.T
    table, ind, loss = _tc_call(x_t, e_t)
    q = _sc_gather(table, ind)
    return (q.reshape(_B0, _S0, _DIM), loss[0, 0], ind)
